# baseline (speedup 1.0000x reference)
import functools

import jax
import jax.numpy as jnp
from jax import lax
from jax.experimental import pallas as pl
from jax.experimental.pallas import tpu as pltpu
from jax.experimental.pallas import tpu_sc as plsc

_FEAT_DIM = 128
_GROWTH = 32
_SCALES = (8, 16, 32)


def _mm_body(w_ref, b_ref, x_ref, o_ref):
    o_ref[...] = (
        jnp.dot(w_ref[...], x_ref[...], preferred_element_type=jnp.float32)
        + b_ref[...]
    )


def _pick_tile(n):
    for t in (2560, 1280, 640, 512, 256, 128):
        if n % t == 0:
            return t
    return n


def _mm_bias(w, b, x):
    o, c = w.shape
    n = x.shape[1]
    t = _pick_tile(n)
    grid = (n // t,)
    return pl.pallas_call(
        _mm_body,
        grid=grid,
        in_specs=[
            pl.BlockSpec((o, c), lambda i: (0, 0)),
            pl.BlockSpec((o, 1), lambda i: (0, 0)),
            pl.BlockSpec((c, t), lambda i: (0, i)),
        ],
        out_specs=pl.BlockSpec((o, t), lambda i: (0, i)),
        out_shape=jax.ShapeDtypeStruct((o, n), jnp.float32),
    )(w, b.reshape(o, 1), x)


def _mmr_body(x_ref, w_ref, b_ref, o_ref):
    o_ref[...] = (
        jnp.dot(x_ref[...], w_ref[...], preferred_element_type=jnp.float32)
        + b_ref[...]
    )


def _mm_rows(x, w, b):
    ln, c = x.shape
    o = w.shape[0]
    t = _pick_tile(ln)
    return pl.pallas_call(
        _mmr_body,
        grid=(ln // t,),
        in_specs=[
            pl.BlockSpec((t, c), lambda i: (i, 0)),
            pl.BlockSpec((c, o), lambda i: (0, 0)),
            pl.BlockSpec((1, o), lambda i: (0, 0)),
        ],
        out_specs=pl.BlockSpec((t, o), lambda i: (i, 0)),
        out_shape=jax.ShapeDtypeStruct((ln, o), jnp.float32),
    )(x, w.T, b.reshape(1, o))


def _bn_relu_rows(y, p):
    mean = jnp.mean(y, axis=0, keepdims=True)
    var = jnp.var(y, axis=0, keepdims=True)
    g = p["gamma"][None, :]
    bt = p["beta"][None, :]
    return jax.nn.relu((y - mean) / jnp.sqrt(var + 1e-5) * g + bt)


def _bn_relu(y, p):
    mean = jnp.mean(y, axis=1, keepdims=True)
    var = jnp.var(y, axis=1, keepdims=True)
    g = p["gamma"][:, None]
    bt = p["beta"][:, None]
    return jax.nn.relu((y - mean) / jnp.sqrt(var + 1e-5) * g + bt)


def _cbr_flat(p, x):
    y = _mm_bias(p["w"], p["b"], x)
    return _bn_relu(y, p)


def _pad_cols(x, n_pad):
    n = x.shape[1]
    if n == n_pad:
        return x
    return jnp.pad(x, ((0, 0), (0, n_pad - n)))


def _cbr1d(p, x):
    c, n = x.shape
    n_pad = ((n + 127) // 128) * 128
    y = _mm_bias(p["w"], p["b"], _pad_cols(x, n_pad))[:, :n]
    return _bn_relu(y, p)


def _gather(feats, idx):
    return feats[:, idx]


_NP = 10240
_NR = 10112
_NG = 320
_GS = 32


def _gmin_body(d_ref, o_ref):
    s = d_ref[...]
    parts = [jnp.min(s[32 * g:32 * (g + 1), :], axis=0, keepdims=True)
             for g in range(_NG)]
    o_ref[...] = jnp.concatenate(parts, axis=0).T


def _group_mins(d2t):
    grid = (_NR // 128,)
    return pl.pallas_call(
        _gmin_body,
        grid=grid,
        in_specs=[pl.BlockSpec((_NP, 128), lambda r: (0, r))],
        out_specs=pl.BlockSpec((128, _NG), lambda r: (r, 0)),
        out_shape=jax.ShapeDtypeStruct((_NR, _NG), jnp.float32),
    )(d2t)


def _sc_merge(bk0, bp0, bk1, bp1, vk, vp):
    vs, ps = plsc.sort_key_val(vk, vp, descending=True)
    m = vs < bk1
    lok = jnp.where(m, vs, bk1)
    lop = jnp.where(m, ps, bp1)
    lks, lps = plsc.sort_key_val(lok, lop)
    rk = lax.rev(lks, (0,))
    rp = lax.rev(lps, (0,))
    m2 = bk0 <= rk
    l2k = jnp.where(m2, bk0, rk)
    l2p = jnp.where(m2, bp0, rp)
    h2k = jnp.where(m2, rk, bk0)
    h2p = jnp.where(m2, rp, bp0)
    nb0 = plsc.sort_key_val(l2k, l2p)
    nb1 = plsc.sort_key_val(h2k, h2p)
    return nb0[0], nb0[1], nb1[0], nb1[1]


def _sc_merge_cond(bt, vk, vp):
    b0k, b0p, b1k, b1p, tau = bt

    def do(op):
        nb = _sc_merge(op[0], op[1], op[2], op[3], vk, vp)
        return (nb[0], nb[1], nb[2], nb[3], jnp.max(nb[2]))

    return lax.cond(jnp.any(vk < tau), do, lambda op: bt,
                    (b0k, b0p, b1k, b1p))


def _sc_init2(k0, p0, k1, p1):
    a0, q0 = plsc.sort_key_val(k0, p0)
    a1, q1 = plsc.sort_key_val(k1, p1)
    rk = lax.rev(a1, (0,))
    rp = lax.rev(q1, (0,))
    m = a0 <= rk
    lk = jnp.where(m, a0, rk)
    lp = jnp.where(m, q0, rp)
    hk = jnp.where(m, rk, a0)
    hp = jnp.where(m, rp, q0)
    b0 = plsc.sort_key_val(lk, lp)
    b1 = plsc.sort_key_val(hk, hp)
    return b0[0], b0[1], b1[0], b1[1]


def _make_sc_knn():
    nc, ns = 2, 16
    nw = nc * ns
    rpw = _NR // nw
    mesh = plsc.VectorSubcoreMesh(core_axis_name="c", subcore_axis_name="s")

    @functools.partial(
        pl.kernel, mesh=mesh,
        out_type=jax.ShapeDtypeStruct((_NR, 32), jnp.int32),
        compiler_params=pltpu.CompilerParams(
            needs_layout_passes=False, use_tc_tiling_on_sc=False),
        scratch_types=[
            pltpu.VMEM((_NG,), jnp.float32),
            pltpu.VMEM((32,), jnp.int32),
            pltpu.VMEM((32,), jnp.int32),
            pltpu.VMEM((32, 32), jnp.float32),
            pltpu.VMEM((32,), jnp.int32),
            pltpu.SemaphoreType.DMA,
        ],
    )
    def knn_sc(m_hbm, tbl_hbm, out_hbm, mrow, gidxv, tidxv, candv,
               orow, sem):
        wid = lax.axis_index("s") * nc + lax.axis_index("c")
        base = wid * rpw
        iota = lax.iota(jnp.int32, 16)

        def row_body(i, carry):
            row = base + i
            pltpu.sync_copy(m_hbm.at[row], mrow)
            b = _sc_init2(mrow[pl.ds(0, 16)], iota,
                          mrow[pl.ds(16, 16)], iota + 16)

            def gb(c, b):
                return _sc_merge(*b, mrow[pl.ds(c * 16, 16)], iota + c * 16)

            b0k, b0p, b1k, b1p = lax.fori_loop(2, _NG // 16, gb, b)
            gidxv[pl.ds(0, 16)] = b0p
            gidxv[pl.ds(16, 16)] = b1p
            tidxv[pl.ds(0, 16)] = b0p + row * _NG
            tidxv[pl.ds(16, 16)] = b1p + row * _NG
            pltpu.async_copy(tbl_hbm.at[tidxv], candv, sem).wait()

            def cand_chunk(j):
                s = candv[j // 2, pl.ds((j % 2) * 16, 16)]
                return s, j * 16 + iota

            s0, c0 = cand_chunk(0)
            s1, c1 = cand_chunk(1)
            bc = _sc_init2(s0, c0, s1, c1)

            def gc(j, bc):
                s, cidx = cand_chunk(j)
                return _sc_merge(*bc, s, cidx)

            f0k, f0p, f1k, f1p = lax.fori_loop(2, 64, gc, bc)
            g0 = plsc.load_gather(gidxv, [lax.shift_right_logical(f0p, 5)])
            g1 = plsc.load_gather(gidxv, [lax.shift_right_logical(f1p, 5)])
            orow[pl.ds(0, 16)] = g0 * 32 + (f0p & 31)
            orow[pl.ds(16, 16)] = g1 * 32 + (f1p & 31)
            pltpu.sync_copy(orow, out_hbm.at[row])
            return carry

        lax.fori_loop(0, rpw, row_body, 0)

    return knn_sc


_GATHER_CACHE = {}


def _make_sc_gather(d, lp):
    nc, ns = 2, 16
    nw = nc * ns
    blk = 512
    cpw = lp // nw
    nb = cpw // blk
    mesh = plsc.VectorSubcoreMesh(core_axis_name="c", subcore_axis_name="s")

    @functools.partial(
        pl.kernel, mesh=mesh,
        out_type=jax.ShapeDtypeStruct((lp, d), jnp.float32),
        compiler_params=pltpu.CompilerParams(
            needs_layout_passes=False, use_tc_tiling_on_sc=False),
        scratch_types=[
            pltpu.VMEM((blk,), jnp.int32),
            pltpu.VMEM((blk, d), jnp.float32),
            pltpu.SemaphoreType.DMA,
        ],
    )
    def gk(tbl_hbm, idx_hbm, out_hbm, idxv, rowsv, sem):
        wid = lax.axis_index("s") * nc + lax.axis_index("c")
        base = wid * cpw

        def body(i, carry):
            off = base + i * blk
            pltpu.sync_copy(idx_hbm.at[pl.ds(off, blk)], idxv)
            pltpu.async_copy(tbl_hbm.at[idxv], rowsv, sem).wait()
            pltpu.sync_copy(rowsv, out_hbm.at[pl.ds(off, blk)])
            return carry

        lax.fori_loop(0, nb, body, 0)

    return gk


def _sc_gather(tbl, idx_flat):
    ln = idx_flat.shape[0]
    d = tbl.shape[1]
    unit = 32 * 512
    lp = ((ln + unit - 1) // unit) * unit
    idx_p = jnp.zeros((lp,), jnp.int32).at[:ln].set(idx_flat)
    key = (d, lp)
    if key not in _GATHER_CACHE:
        _GATHER_CACHE[key] = _make_sc_gather(d, lp)
    return _GATHER_CACHE[key](tbl, idx_p)[:ln]


def _knn_pallas(pts):
    n = pts.shape[2]
    ptst = jnp.transpose(pts, (0, 2, 1))
    sq = jnp.sum(ptst ** 2, axis=-1)
    prow = jnp.concatenate(
        [ptst, jnp.zeros((1, _NR - n, 3), jnp.float32)], axis=1)
    pcol = jnp.concatenate(
        [ptst, jnp.zeros((1, _NP - n, 3), jnp.float32)], axis=1)
    sqrow = jnp.concatenate(
        [sq, jnp.zeros((1, _NR - n), jnp.float32)], axis=1)
    sqcol = jnp.concatenate(
        [sq, jnp.full((1, _NP - n), 1e30, jnp.float32)], axis=1)
    dots_n = jnp.einsum('bnc,bmc->bnm', prow, pcol)
    d2n = sqrow[:, :, None] + sqcol[:, None, :] - 2.0 * dots_n
    dots_t = jnp.einsum('bnc,bmc->bnm', pcol, prow)
    d2t = sqcol[:, :, None] + sqrow[:, None, :] - 2.0 * dots_t
    m = _group_mins(d2t[0])
    tbl = d2n[0].reshape(_NR * _NG, _GS)
    idx = _make_sc_knn()(m, tbl)
    return idx[:n]


def kernel(pts, params):
    scales = _SCALES
    n = pts.shape[2]
    idx_all = _knn_pallas(pts)
    p2 = pts[0]

    tp = jnp.zeros((_NP, 16), jnp.float32).at[:n, :3].set(p2.T)

    init_feats = _cbr1d(params["conv_init"], p2)

    scale_features_all = []
    for s in range(len(scales)):
        k = scales[s]
        idx_flat = idx_all[:, :k].reshape(-1)
        tpg = _sc_gather(tp, idx_flat)
        delta = (tpg[:, :3].reshape(n, k, 3)
                 - p2.T[:, None, :]).reshape(n * k, 3)
        feats = init_feats
        locs = [feats]
        for blk in params["scales"][s]:
            for lp in blk["dense"]:
                nf = _cbr1d(lp["bottle"], feats)
                pd = lp["pc"]["delta"]
                d = _bn_relu_rows(_mm_rows(delta, pd["w"], pd["b"]), pd)
                kfp = _mm_bias(lp["pc"]["feats"]["w"],
                               lp["pc"]["feats"]["b"],
                               _pad_cols(nf, _NP))
                tk = kfp.T
                kf = _sc_gather(tk, idx_flat)
                kf = _bn_relu_rows(kf, lp["pc"]["feats"])
                pp_ = lp["pc"]["post"]
                nf2 = _bn_relu_rows(_mm_rows(d * kf, pp_["w"], pp_["b"]), pp_)
                nf2 = jnp.sum(nf2.reshape(n, k, _GROWTH), axis=1).T
                feats = jnp.concatenate([feats, nf2], axis=0)
            feats = _cbr1d(blk["trans"], feats)
            locs.append(feats)
        scale_features_all.append(locs)

    fused_list = []
    for bi in range(len(scale_features_all[0])):
        f = scale_features_all[0][bi]
        for s in range(1, len(scales)):
            cat = jnp.concatenate([f, scale_features_all[s][bi]], axis=0)
            f = _cbr1d(params["fusion"][s - 1], cat)
        fused_list.append(f)
    local_stack = jnp.stack(fused_list, axis=0)[:, None]
    global_feats = jnp.max(local_stack[-1], axis=-1)
    return (global_feats, local_stack)


# speedup vs baseline: 1.0843x; 1.0843x over previous
import functools

import jax
import jax.numpy as jnp
from jax import lax
from jax.experimental import pallas as pl
from jax.experimental.pallas import tpu as pltpu
from jax.experimental.pallas import tpu_sc as plsc

_FEAT_DIM = 128
_GROWTH = 32
_SCALES = (8, 16, 32)


def _mm_body(w_ref, b_ref, x_ref, o_ref):
    o_ref[...] = (
        jnp.dot(w_ref[...], x_ref[...], preferred_element_type=jnp.float32)
        + b_ref[...]
    )


def _pick_tile(n):
    for t in (2560, 1280, 640, 512, 256, 128):
        if n % t == 0:
            return t
    return n


def _mm_bias(w, b, x):
    o, c = w.shape
    n = x.shape[1]
    t = _pick_tile(n)
    grid = (n // t,)
    return pl.pallas_call(
        _mm_body,
        grid=grid,
        in_specs=[
            pl.BlockSpec((o, c), lambda i: (0, 0)),
            pl.BlockSpec((o, 1), lambda i: (0, 0)),
            pl.BlockSpec((c, t), lambda i: (0, i)),
        ],
        out_specs=pl.BlockSpec((o, t), lambda i: (0, i)),
        out_shape=jax.ShapeDtypeStruct((o, n), jnp.float32),
    )(w, b.reshape(o, 1), x)


def _mmr_body(x_ref, w_ref, b_ref, o_ref):
    o_ref[...] = (
        jnp.dot(x_ref[...], w_ref[...], preferred_element_type=jnp.float32)
        + b_ref[...]
    )


def _mm_rows(x, w, b):
    ln, c = x.shape
    o = w.shape[0]
    t = _pick_tile(ln)
    return pl.pallas_call(
        _mmr_body,
        grid=(ln // t,),
        in_specs=[
            pl.BlockSpec((t, c), lambda i: (i, 0)),
            pl.BlockSpec((c, o), lambda i: (0, 0)),
            pl.BlockSpec((1, o), lambda i: (0, 0)),
        ],
        out_specs=pl.BlockSpec((t, o), lambda i: (i, 0)),
        out_shape=jax.ShapeDtypeStruct((ln, o), jnp.float32),
    )(x, w.T, b.reshape(1, o))


def _bn_relu_rows(y, p):
    mean = jnp.mean(y, axis=0, keepdims=True)
    var = jnp.var(y, axis=0, keepdims=True)
    g = p["gamma"][None, :]
    bt = p["beta"][None, :]
    return jax.nn.relu((y - mean) / jnp.sqrt(var + 1e-5) * g + bt)


def _t_body(x_ref, o_ref):
    o_ref[...] = x_ref[...].T


def _pl_t(x):
    ln = x.shape[1 - 1]
    t = _pick_tile(ln)
    return pl.pallas_call(
        _t_body,
        grid=(ln // t,),
        in_specs=[pl.BlockSpec((t, 32), lambda i: (i, 0))],
        out_specs=pl.BlockSpec((32, t), lambda i: (0, i)),
        out_shape=jax.ShapeDtypeStruct((32, ln), jnp.float32),
    )(x)


def _bn_relu(y, p):
    mean = jnp.mean(y, axis=1, keepdims=True)
    var = jnp.var(y, axis=1, keepdims=True)
    g = p["gamma"][:, None]
    bt = p["beta"][:, None]
    return jax.nn.relu((y - mean) / jnp.sqrt(var + 1e-5) * g + bt)


def _cbr_flat(p, x):
    y = _mm_bias(p["w"], p["b"], x)
    return _bn_relu(y, p)


def _pad_cols(x, n_pad):
    n = x.shape[1]
    if n == n_pad:
        return x
    return jnp.pad(x, ((0, 0), (0, n_pad - n)))


def _cbr1d(p, x):
    c, n = x.shape
    n_pad = ((n + 127) // 128) * 128
    y = _mm_bias(p["w"], p["b"], _pad_cols(x, n_pad))[:, :n]
    return _bn_relu(y, p)


def _gather(feats, idx):
    return feats[:, idx]


_NP = 10240
_NR = 10112
_NG = 320
_GS = 32


def _gmin_body(d_ref, o_ref):
    s = d_ref[...]
    parts = [jnp.min(s[32 * g:32 * (g + 1), :], axis=0, keepdims=True)
             for g in range(_NG)]
    o_ref[...] = jnp.concatenate(parts, axis=0).T


def _group_mins(d2t):
    grid = (_NR // 128,)
    return pl.pallas_call(
        _gmin_body,
        grid=grid,
        in_specs=[pl.BlockSpec((_NP, 128), lambda r: (0, r))],
        out_specs=pl.BlockSpec((128, _NG), lambda r: (r, 0)),
        out_shape=jax.ShapeDtypeStruct((_NR, _NG), jnp.float32),
    )(d2t)


def _sc_merge(bk0, bp0, bk1, bp1, vk, vp):
    vs, ps = plsc.sort_key_val(vk, vp, descending=True)
    m = vs < bk1
    lok = jnp.where(m, vs, bk1)
    lop = jnp.where(m, ps, bp1)
    lks, lps = plsc.sort_key_val(lok, lop)
    rk = lax.rev(lks, (0,))
    rp = lax.rev(lps, (0,))
    m2 = bk0 <= rk
    l2k = jnp.where(m2, bk0, rk)
    l2p = jnp.where(m2, bp0, rp)
    h2k = jnp.where(m2, rk, bk0)
    h2p = jnp.where(m2, rp, bp0)
    nb0 = plsc.sort_key_val(l2k, l2p)
    nb1 = plsc.sort_key_val(h2k, h2p)
    return nb0[0], nb0[1], nb1[0], nb1[1]


def _sc_merge_cond(bt, vk, vp):
    b0k, b0p, b1k, b1p, tau = bt

    def do(op):
        nb = _sc_merge(op[0], op[1], op[2], op[3], vk, vp)
        return (nb[0], nb[1], nb[2], nb[3], jnp.max(nb[2]))

    return lax.cond(jnp.any(vk < tau), do, lambda op: bt,
                    (b0k, b0p, b1k, b1p))


def _sc_init2(k0, p0, k1, p1):
    a0, q0 = plsc.sort_key_val(k0, p0)
    a1, q1 = plsc.sort_key_val(k1, p1)
    rk = lax.rev(a1, (0,))
    rp = lax.rev(q1, (0,))
    m = a0 <= rk
    lk = jnp.where(m, a0, rk)
    lp = jnp.where(m, q0, rp)
    hk = jnp.where(m, rk, a0)
    hp = jnp.where(m, rp, q0)
    b0 = plsc.sort_key_val(lk, lp)
    b1 = plsc.sort_key_val(hk, hp)
    return b0[0], b0[1], b1[0], b1[1]


def _make_sc_knn():
    nc, ns = 2, 16
    nw = nc * ns
    rpw = _NR // nw
    mesh = plsc.VectorSubcoreMesh(core_axis_name="c", subcore_axis_name="s")

    @functools.partial(
        pl.kernel, mesh=mesh,
        out_type=jax.ShapeDtypeStruct((_NR, 32), jnp.int32),
        compiler_params=pltpu.CompilerParams(
            needs_layout_passes=False, use_tc_tiling_on_sc=False),
        scratch_types=[
            pltpu.VMEM((_NG,), jnp.float32),
            pltpu.VMEM((32,), jnp.int32),
            pltpu.VMEM((32,), jnp.int32),
            pltpu.VMEM((32, 32), jnp.float32),
            pltpu.VMEM((32,), jnp.int32),
            pltpu.SemaphoreType.DMA,
        ],
    )
    def knn_sc(m_hbm, tbl_hbm, out_hbm, mrow, gidxv, tidxv, candv,
               orow, sem):
        wid = lax.axis_index("s") * nc + lax.axis_index("c")
        base = wid * rpw
        iota = lax.iota(jnp.int32, 16)

        def row_body(i, carry):
            row = base + i
            pltpu.sync_copy(m_hbm.at[row], mrow)
            b = _sc_init2(mrow[pl.ds(0, 16)], iota,
                          mrow[pl.ds(16, 16)], iota + 16)

            def gb(c, b):
                return _sc_merge(*b, mrow[pl.ds(c * 16, 16)], iota + c * 16)

            b0k, b0p, b1k, b1p = lax.fori_loop(2, _NG // 16, gb, b)
            gidxv[pl.ds(0, 16)] = b0p
            gidxv[pl.ds(16, 16)] = b1p
            tidxv[pl.ds(0, 16)] = b0p + row * _NG
            tidxv[pl.ds(16, 16)] = b1p + row * _NG
            pltpu.async_copy(tbl_hbm.at[tidxv], candv, sem).wait()

            def cand_chunk(j):
                s = candv[j // 2, pl.ds((j % 2) * 16, 16)]
                return s, j * 16 + iota

            s0, c0 = cand_chunk(0)
            s1, c1 = cand_chunk(1)
            bc = _sc_init2(s0, c0, s1, c1)

            def gc(j, bc):
                s, cidx = cand_chunk(j)
                return _sc_merge(*bc, s, cidx)

            f0k, f0p, f1k, f1p = lax.fori_loop(2, 64, gc, bc)
            g0 = plsc.load_gather(gidxv, [lax.shift_right_logical(f0p, 5)])
            g1 = plsc.load_gather(gidxv, [lax.shift_right_logical(f1p, 5)])
            orow[pl.ds(0, 16)] = g0 * 32 + (f0p & 31)
            orow[pl.ds(16, 16)] = g1 * 32 + (f1p & 31)
            pltpu.sync_copy(orow, out_hbm.at[row])
            return carry

        lax.fori_loop(0, rpw, row_body, 0)

    return knn_sc


_GATHER_CACHE = {}


def _make_sc_gather(d, lp):
    nc, ns = 2, 16
    nw = nc * ns
    blk = 512
    cpw = lp // nw
    nb = cpw // blk
    mesh = plsc.VectorSubcoreMesh(core_axis_name="c", subcore_axis_name="s")

    @functools.partial(
        pl.kernel, mesh=mesh,
        out_type=jax.ShapeDtypeStruct((lp, d), jnp.float32),
        compiler_params=pltpu.CompilerParams(
            needs_layout_passes=False, use_tc_tiling_on_sc=False),
        scratch_types=[
            pltpu.VMEM((blk,), jnp.int32),
            pltpu.VMEM((blk, d), jnp.float32),
            pltpu.SemaphoreType.DMA,
        ],
    )
    def gk(tbl_hbm, idx_hbm, out_hbm, idxv, rowsv, sem):
        wid = lax.axis_index("s") * nc + lax.axis_index("c")
        base = wid * cpw

        def body(i, carry):
            off = base + i * blk
            pltpu.sync_copy(idx_hbm.at[pl.ds(off, blk)], idxv)
            pltpu.async_copy(tbl_hbm.at[idxv], rowsv, sem).wait()
            pltpu.sync_copy(rowsv, out_hbm.at[pl.ds(off, blk)])
            return carry

        lax.fori_loop(0, nb, body, 0)

    return gk


def _sc_gather(tbl, idx_flat):
    ln = idx_flat.shape[0]
    d = tbl.shape[1]
    unit = 32 * 512
    lp = ((ln + unit - 1) // unit) * unit
    idx_p = jnp.zeros((lp,), jnp.int32).at[:ln].set(idx_flat)
    key = (d, lp)
    if key not in _GATHER_CACHE:
        _GATHER_CACHE[key] = _make_sc_gather(d, lp)
    return _GATHER_CACHE[key](tbl, idx_p)[:ln]


def _knn_pallas(pts):
    n = pts.shape[2]
    ptst = jnp.transpose(pts, (0, 2, 1))
    sq = jnp.sum(ptst ** 2, axis=-1)
    prow = jnp.concatenate(
        [ptst, jnp.zeros((1, _NR - n, 3), jnp.float32)], axis=1)
    pcol = jnp.concatenate(
        [ptst, jnp.zeros((1, _NP - n, 3), jnp.float32)], axis=1)
    sqrow = jnp.concatenate(
        [sq, jnp.zeros((1, _NR - n), jnp.float32)], axis=1)
    sqcol = jnp.concatenate(
        [sq, jnp.full((1, _NP - n), 1e30, jnp.float32)], axis=1)
    dots_n = jnp.einsum('bnc,bmc->bnm', prow, pcol)
    d2n = sqrow[:, :, None] + sqcol[:, None, :] - 2.0 * dots_n
    dots_t = jnp.einsum('bnc,bmc->bnm', pcol, prow)
    d2t = sqcol[:, :, None] + sqrow[:, None, :] - 2.0 * dots_t
    m = _group_mins(d2t[0])
    tbl = d2n[0].reshape(_NR * _NG, _GS)
    idx = _make_sc_knn()(m, tbl)
    return idx[:n]


def kernel(pts, params):
    scales = _SCALES
    n = pts.shape[2]
    idx_all = _knn_pallas(pts)
    p2 = pts[0]

    tp = jnp.zeros((_NP, 16), jnp.float32).at[:n, :3].set(p2.T)

    init_feats = _cbr1d(params["conv_init"], p2)

    scale_features_all = []
    for s in range(len(scales)):
        k = scales[s]
        idx_flat = idx_all[:, :k].reshape(-1)
        tpg = _sc_gather(tp, idx_flat)
        delta = (tpg[:, :3].reshape(n, k, 3) - p2.T[:, None, :])
        delta = delta.transpose(2, 0, 1).reshape(3, n * k)
        feats = init_feats
        locs = [feats]
        for blk in params["scales"][s]:
            for lp in blk["dense"]:
                nf = _cbr1d(lp["bottle"], feats)
                d = _cbr_flat(lp["pc"]["delta"], delta)
                kfp = _mm_bias(lp["pc"]["feats"]["w"],
                               lp["pc"]["feats"]["b"],
                               _pad_cols(nf, _NP))
                tk = kfp.T
                kf = _pl_t(_sc_gather(tk, idx_flat))
                kf = _bn_relu(kf, lp["pc"]["feats"])
                nf2 = _cbr_flat(lp["pc"]["post"], d * kf)
                nf2 = jnp.sum(nf2.reshape(_GROWTH, n, k), axis=-1)
                feats = jnp.concatenate([feats, nf2], axis=0)
            feats = _cbr1d(blk["trans"], feats)
            locs.append(feats)
        scale_features_all.append(locs)

    fused_list = []
    for bi in range(len(scale_features_all[0])):
        f = scale_features_all[0][bi]
        for s in range(1, len(scales)):
            cat = jnp.concatenate([f, scale_features_all[s][bi]], axis=0)
            f = _cbr1d(params["fusion"][s - 1], cat)
        fused_list.append(f)
    local_stack = jnp.stack(fused_list, axis=0)[:, None]
    global_feats = jnp.max(local_stack[-1], axis=-1)
    return (global_feats, local_stack)


# speedup vs baseline: 1.1252x; 1.0378x over previous
import functools

import jax
import jax.numpy as jnp
from jax import lax
from jax.experimental import pallas as pl
from jax.experimental.pallas import tpu as pltpu
from jax.experimental.pallas import tpu_sc as plsc

_FEAT_DIM = 128
_GROWTH = 32
_SCALES = (8, 16, 32)


def _mm_body(w_ref, b_ref, x_ref, o_ref):
    o_ref[...] = (
        jnp.dot(w_ref[...], x_ref[...], preferred_element_type=jnp.float32)
        + b_ref[...]
    )


def _pick_tile(n):
    for t in (2560, 1280, 640, 512, 256, 128):
        if n % t == 0:
            return t
    return n


def _mm_bias(w, b, x):
    o, c = w.shape
    n = x.shape[1]
    t = _pick_tile(n)
    grid = (n // t,)
    return pl.pallas_call(
        _mm_body,
        grid=grid,
        in_specs=[
            pl.BlockSpec((o, c), lambda i: (0, 0)),
            pl.BlockSpec((o, 1), lambda i: (0, 0)),
            pl.BlockSpec((c, t), lambda i: (0, i)),
        ],
        out_specs=pl.BlockSpec((o, t), lambda i: (0, i)),
        out_shape=jax.ShapeDtypeStruct((o, n), jnp.float32),
    )(w, b.reshape(o, 1), x)


def _bn_relu(y, p):
    mean = jnp.mean(y, axis=1, keepdims=True)
    var = jnp.var(y, axis=1, keepdims=True)
    g = p["gamma"][:, None]
    bt = p["beta"][:, None]
    return jax.nn.relu((y - mean) / jnp.sqrt(var + 1e-5) * g + bt)


def _cbr_flat(p, x):
    y = _mm_bias(p["w"], p["b"], x)
    return _bn_relu(y, p)


def _pad_cols(x, n_pad):
    n = x.shape[1]
    if n == n_pad:
        return x
    return jnp.pad(x, ((0, 0), (0, n_pad - n)))


def _cbr1d(p, x):
    c, n = x.shape
    n_pad = ((n + 127) // 128) * 128
    y = _mm_bias(p["w"], p["b"], _pad_cols(x, n_pad))[:, :n]
    return _bn_relu(y, p)


_NP = 10240
_NR = 10112
_NG = 320
_GS = 32


def _gmin_body(d_ref, o_ref):
    s = d_ref[...]
    parts = [jnp.min(s[32 * g:32 * (g + 1), :], axis=0, keepdims=True)
             for g in range(_NG)]
    o_ref[...] = jnp.concatenate(parts, axis=0).T


def _group_mins(d2t):
    grid = (_NR // 128,)
    return pl.pallas_call(
        _gmin_body,
        grid=grid,
        in_specs=[pl.BlockSpec((_NP, 128), lambda r: (0, r))],
        out_specs=pl.BlockSpec((128, _NG), lambda r: (r, 0)),
        out_shape=jax.ShapeDtypeStruct((_NR, _NG), jnp.float32),
    )(d2t)


def _sc_merge(bk0, bp0, bk1, bp1, vk, vp):
    vs, ps = plsc.sort_key_val(vk, vp, descending=True)
    m = vs < bk1
    lok = jnp.where(m, vs, bk1)
    lop = jnp.where(m, ps, bp1)
    lks, lps = plsc.sort_key_val(lok, lop)
    rk = lax.rev(lks, (0,))
    rp = lax.rev(lps, (0,))
    m2 = bk0 <= rk
    l2k = jnp.where(m2, bk0, rk)
    l2p = jnp.where(m2, bp0, rp)
    h2k = jnp.where(m2, rk, bk0)
    h2p = jnp.where(m2, rp, bp0)
    nb0 = plsc.sort_key_val(l2k, l2p)
    nb1 = plsc.sort_key_val(h2k, h2p)
    return nb0[0], nb0[1], nb1[0], nb1[1]


def _sc_init2(k0, p0, k1, p1):
    a0, q0 = plsc.sort_key_val(k0, p0)
    a1, q1 = plsc.sort_key_val(k1, p1)
    rk = lax.rev(a1, (0,))
    rp = lax.rev(q1, (0,))
    m = a0 <= rk
    lk = jnp.where(m, a0, rk)
    lp = jnp.where(m, q0, rp)
    hk = jnp.where(m, rk, a0)
    hp = jnp.where(m, rp, q0)
    b0 = plsc.sort_key_val(lk, lp)
    b1 = plsc.sort_key_val(hk, hp)
    return b0[0], b0[1], b1[0], b1[1]


def _make_sc_knn():
    nc, ns = 2, 16
    nw = nc * ns
    rpw = _NR // nw
    mesh = plsc.VectorSubcoreMesh(core_axis_name="c", subcore_axis_name="s")

    @functools.partial(
        pl.kernel, mesh=mesh,
        out_type=jax.ShapeDtypeStruct((_NR, 32), jnp.int32),
        compiler_params=pltpu.CompilerParams(
            needs_layout_passes=False, use_tc_tiling_on_sc=False),
        scratch_types=[
            pltpu.VMEM((_NG,), jnp.float32),
            pltpu.VMEM((32,), jnp.int32),
            pltpu.VMEM((32,), jnp.int32),
            pltpu.VMEM((32, 32), jnp.float32),
            pltpu.VMEM((32,), jnp.int32),
            pltpu.SemaphoreType.DMA,
        ],
    )
    def knn_sc(m_hbm, tbl_hbm, out_hbm, mrow, gidxv, tidxv, candv,
               orow, sem):
        wid = lax.axis_index("s") * nc + lax.axis_index("c")
        base = wid * rpw
        iota = lax.iota(jnp.int32, 16)

        def row_body(i, carry):
            row = base + i
            pltpu.sync_copy(m_hbm.at[row], mrow)
            b = _sc_init2(mrow[pl.ds(0, 16)], iota,
                          mrow[pl.ds(16, 16)], iota + 16)

            def gb(c, b):
                return _sc_merge(*b, mrow[pl.ds(c * 16, 16)], iota + c * 16)

            b0k, b0p, b1k, b1p = lax.fori_loop(2, _NG // 16, gb, b)
            gidxv[pl.ds(0, 16)] = b0p
            gidxv[pl.ds(16, 16)] = b1p
            tidxv[pl.ds(0, 16)] = b0p + row * _NG
            tidxv[pl.ds(16, 16)] = b1p + row * _NG
            pltpu.async_copy(tbl_hbm.at[tidxv], candv, sem).wait()

            def cand_chunk(j):
                s = candv[j // 2, pl.ds((j % 2) * 16, 16)]
                return s, j * 16 + iota

            s0, c0 = cand_chunk(0)
            s1, c1 = cand_chunk(1)
            bc = _sc_init2(s0, c0, s1, c1)

            def gc(j, bc):
                s, cidx = cand_chunk(j)
                return _sc_merge(*bc, s, cidx)

            f0k, f0p, f1k, f1p = lax.fori_loop(2, 64, gc, bc)
            g0 = plsc.load_gather(gidxv, [lax.shift_right_logical(f0p, 5)])
            g1 = plsc.load_gather(gidxv, [lax.shift_right_logical(f1p, 5)])
            orow[pl.ds(0, 16)] = g0 * 32 + (f0p & 31)
            orow[pl.ds(16, 16)] = g1 * 32 + (f1p & 31)
            pltpu.sync_copy(orow, out_hbm.at[row])
            return carry

        lax.fori_loop(0, rpw, row_body, 0)

    return knn_sc


_GATHER_CACHE = {}


def _make_sc_gather(d, lp):
    nc, ns = 2, 16
    nw = nc * ns
    blk = 512
    cpw = lp // nw
    nb = cpw // blk
    mesh = plsc.VectorSubcoreMesh(core_axis_name="c", subcore_axis_name="s")

    @functools.partial(
        pl.kernel, mesh=mesh,
        out_type=jax.ShapeDtypeStruct((lp, d), jnp.float32),
        compiler_params=pltpu.CompilerParams(
            needs_layout_passes=False, use_tc_tiling_on_sc=False),
        scratch_types=[
            pltpu.VMEM((blk,), jnp.int32),
            pltpu.VMEM((blk, d), jnp.float32),
            pltpu.SemaphoreType.DMA,
        ],
    )
    def gk(tbl_hbm, idx_hbm, out_hbm, idxv, rowsv, sem):
        wid = lax.axis_index("s") * nc + lax.axis_index("c")
        base = wid * cpw

        def body(i, carry):
            off = base + i * blk
            pltpu.sync_copy(idx_hbm.at[pl.ds(off, blk)], idxv)
            pltpu.async_copy(tbl_hbm.at[idxv], rowsv, sem).wait()
            pltpu.sync_copy(rowsv, out_hbm.at[pl.ds(off, blk)])
            return carry

        lax.fori_loop(0, nb, body, 0)

    return gk


def _sc_gather(tbl, idx_flat):
    ln = idx_flat.shape[0]
    d = tbl.shape[1]
    unit = 32 * 512
    lp = ((ln + unit - 1) // unit) * unit
    idx_p = jnp.zeros((lp,), jnp.int32).at[:ln].set(idx_flat)
    key = (d, lp)
    if key not in _GATHER_CACHE:
        _GATHER_CACHE[key] = _make_sc_gather(d, lp)
    return _GATHER_CACHE[key](tbl, idx_p)[:ln]


def _knn_pallas(pts):
    n = pts.shape[2]
    ptst = jnp.transpose(pts, (0, 2, 1))
    sq = jnp.sum(ptst ** 2, axis=-1)
    prow = jnp.concatenate(
        [ptst, jnp.zeros((1, _NR - n, 3), jnp.float32)], axis=1)
    pcol = jnp.concatenate(
        [ptst, jnp.zeros((1, _NP - n, 3), jnp.float32)], axis=1)
    sqrow = jnp.concatenate(
        [sq, jnp.zeros((1, _NR - n), jnp.float32)], axis=1)
    sqcol = jnp.concatenate(
        [sq, jnp.full((1, _NP - n), 1e30, jnp.float32)], axis=1)
    dots_n = jnp.einsum('bnc,bmc->bnm', prow, pcol)
    d2n = sqrow[:, :, None] + sqcol[:, None, :] - 2.0 * dots_n
    dots_t = jnp.einsum('bnc,bmc->bnm', pcol, prow)
    d2t = sqcol[:, :, None] + sqrow[:, None, :] - 2.0 * dots_t
    m = _group_mins(d2t[0])
    tbl = d2n[0].reshape(_NR * _NG, _GS)
    idx = _make_sc_knn()(m, tbl)
    return idx[:n]


def kernel(pts, params):
    scales = _SCALES
    n = pts.shape[2]
    idx_all = _knn_pallas(pts)
    p2 = pts[0]

    tp = jnp.zeros((_NP, 16), jnp.float32).at[:n, :3].set(p2.T)

    init_feats = _cbr1d(params["conv_init"], p2)

    scale_features_all = []
    for s in range(len(scales)):
        k = scales[s]
        idx_flat = idx_all[:, :k].reshape(-1)
        tpg = _sc_gather(tp, idx_flat)
        delta = (tpg[:, :3].reshape(n, k, 3) - p2.T[:, None, :])
        delta = delta.transpose(2, 0, 1).reshape(3, n * k)
        feats = init_feats
        locs = [feats]
        for blk in params["scales"][s]:
            for lp in blk["dense"]:
                nf = _cbr1d(lp["bottle"], feats)
                d = _cbr_flat(lp["pc"]["delta"], delta)
                kfp = _mm_bias(lp["pc"]["feats"]["w"],
                               lp["pc"]["feats"]["b"],
                               _pad_cols(nf, _NP))
                tk = kfp.T
                kf = _sc_gather(tk, idx_flat).T
                kf = _bn_relu(kf, lp["pc"]["feats"])
                nf2 = _cbr_flat(lp["pc"]["post"], d * kf)
                nf2 = jnp.sum(nf2.reshape(_GROWTH, n, k), axis=-1)
                feats = jnp.concatenate([feats, nf2], axis=0)
            feats = _cbr1d(blk["trans"], feats)
            locs.append(feats)
        scale_features_all.append(locs)

    fused_list = []
    for bi in range(len(scale_features_all[0])):
        f = scale_features_all[0][bi]
        for s in range(1, len(scales)):
            cat = jnp.concatenate([f, scale_features_all[s][bi]], axis=0)
            f = _cbr1d(params["fusion"][s - 1], cat)
        fused_list.append(f)
    local_stack = jnp.stack(fused_list, axis=0)[:, None]
    global_feats = jnp.max(local_stack[-1], axis=-1)
    return (global_feats, local_stack)
